# baseline (speedup 1.0000x reference)
import functools

import jax
import jax.numpy as jnp
from jax import lax
from jax.experimental import pallas as pl
from jax.experimental.pallas import tpu as pltpu
from jax.experimental.pallas import tpu_sc as plsc

SEQ_LEN = 20
BATCH = 4096
N_EMB = 26
VOCAB = 100000
DIM = 32

NTOT = SEQ_LEN * BATCH * N_EMB
NW = 32
PER_W = NTOT // NW
PAT = 208
CHUNK = 832
SUB = 104
NSUB = CHUNK // SUB
NCHUNK = PER_W // CHUNK
assert PER_W % CHUNK == 0 and CHUNK % PAT == 0 and CHUNK % SUB == 0
assert NCHUNK >= 4 and NCHUNK % 2 == 0


def _body(idx_hbm, tbl_hbm, out_hbm, offs_v, idx_v0, idx_v1, rows_v0,
          rows_v1, isem0, isem1, gsem0, gsem1, osem0, osem1):
    wid = lax.axis_index("s") * 2 + lax.axis_index("c")
    idx_b = (idx_v0, idx_v1)
    rows_b = (rows_v0, rows_v1)
    isem = (isem0, isem1)
    gsem = (gsem0, gsem1)
    osem = (osem0, osem1)

    for i in range(PAT // 16):
        v = lax.iota(jnp.int32, 16) + (i * 16)
        offs_v[pl.ds(i * 16, 16)] = lax.rem(v, N_EMB) * VOCAB

    def base(c):
        return wid * PER_W + c * CHUNK

    def fire_gathers(b):
        for k in range(NSUB):
            pltpu.async_copy(
                tbl_hbm.at[idx_b[b].at[pl.ds(k * SUB, SUB)]],
                rows_b[b].at[pl.ds(k * SUB, SUB)],
                gsem[b])

    def drain_gathers(b):
        for k in range(NSUB):
            pltpu.make_async_copy(
                tbl_hbm.at[idx_b[b].at[pl.ds(k * SUB, SUB)]],
                rows_b[b].at[pl.ds(k * SUB, SUB)],
                gsem[b]).wait()

    def add_offsets(b):
        for j in range(CHUNK // 16):
            s = j * 16
            p = (j % (PAT // 16)) * 16
            idx_b[b][pl.ds(s, 16)] = (
                idx_b[b][pl.ds(s, 16)] + offs_v[pl.ds(p, 16)])

    def pipe_iter(c, b, has_prev, has_prev2, has_next):
        b1 = 1 - b
        if has_prev:
            drain_gathers(b1)
            pltpu.async_copy(rows_b[b1],
                             out_hbm.at[pl.ds(base(c - 1), CHUNK)], osem[b1])
        if has_next:
            pltpu.async_copy(idx_hbm.at[pl.ds(base(c + 1), CHUNK)],
                             idx_b[b1], isem[b1])
        pltpu.make_async_copy(idx_hbm.at[pl.ds(0, CHUNK)],
                              idx_b[b], isem[b]).wait()
        add_offsets(b)
        if has_prev2:
            pltpu.make_async_copy(rows_b[b], out_hbm.at[pl.ds(0, CHUNK)],
                                  osem[b]).wait()
        fire_gathers(b)

    pltpu.async_copy(idx_hbm.at[pl.ds(base(0), CHUNK)], idx_b[0], isem[0])
    pipe_iter(0, 0, False, False, True)
    pipe_iter(1, 1, True, False, True)

    def pair_body(t, carry):
        pipe_iter(2 * t, 0, True, True, True)
        pipe_iter(2 * t + 1, 1, True, True, True)
        return carry

    lax.fori_loop(1, NCHUNK // 2 - 1, pair_body, 0)

    pipe_iter(NCHUNK - 2, 0, True, True, True)
    pipe_iter(NCHUNK - 1, 1, True, True, False)

    drain_gathers(1)
    pltpu.async_copy(rows_b[1], out_hbm.at[pl.ds(base(NCHUNK - 1), CHUNK)],
                     osem[1])
    pltpu.make_async_copy(rows_b[0], out_hbm.at[pl.ds(0, CHUNK)],
                          osem[0]).wait()
    pltpu.make_async_copy(rows_b[1], out_hbm.at[pl.ds(0, CHUNK)],
                          osem[1]).wait()


@jax.jit
def _multi_embedding(idx_flat, tbl_flat):
    mesh = plsc.VectorSubcoreMesh(core_axis_name="c", subcore_axis_name="s")
    return pl.kernel(
        _body,
        out_type=jax.ShapeDtypeStruct((NTOT, DIM), jnp.float32),
        mesh=mesh,
        scratch_types=[
            pltpu.VMEM((PAT,), jnp.int32),
            pltpu.VMEM((CHUNK,), jnp.int32),
            pltpu.VMEM((CHUNK,), jnp.int32),
            pltpu.VMEM((CHUNK, DIM), jnp.float32),
            pltpu.VMEM((CHUNK, DIM), jnp.float32),
            pltpu.SemaphoreType.DMA,
            pltpu.SemaphoreType.DMA,
            pltpu.SemaphoreType.DMA,
            pltpu.SemaphoreType.DMA,
            pltpu.SemaphoreType.DMA,
            pltpu.SemaphoreType.DMA,
        ],
        compiler_params=pltpu.CompilerParams(use_tc_tiling_on_sc=False),
    )(idx_flat, tbl_flat)


FEAT = N_EMB * DIM


TCB = 4


def _transpose_body(x_ref, o_ref):
    m = lax.broadcasted_iota(jnp.int32, (64, 128), 0)
    c = lax.broadcasted_iota(jnp.int32, (64, 128), 1)
    p_ev = (c == 2 * m).astype(jnp.float32)
    p_od = (c == 2 * m + 1).astype(jnp.float32)
    dn = (((0,), (0,)), ((), ()))
    for k in range(TCB):
        x3 = x_ref[pl.ds(k * FEAT, FEAT), :].reshape(64, 13, 128)
        ev = jnp.concatenate(
            [x3[:, 0:6, :].reshape(64, 768), x3[:, 6, 0:64]], axis=1)
        od = jnp.concatenate(
            [x3[:, 6, 64:128], x3[:, 7:13, :].reshape(64, 768)], axis=1)
        o_ref[0, :, pl.ds(k * 128, 128)] = (
            lax.dot_general(ev, p_ev, dn, preferred_element_type=jnp.float32)
            + lax.dot_general(od, p_od, dn, preferred_element_type=jnp.float32))


@jax.jit
def _to_feature_major(x128):
    return pl.pallas_call(
        _transpose_body,
        grid=(SEQ_LEN, BATCH // (128 * TCB)),
        in_specs=[pl.BlockSpec((TCB * FEAT, 128),
                               lambda s, bb: (s * (BATCH // (128 * TCB)) + bb, 0))],
        out_specs=pl.BlockSpec((1, FEAT, TCB * 128), lambda s, bb: (s, 0, bb)),
        out_shape=jax.ShapeDtypeStruct((SEQ_LEN, FEAT, BATCH), jnp.float32),
    )(x128)


V2 = 2500
VI = 8


def _table_body(t_ref, y_ref):
    eye = (lax.broadcasted_iota(jnp.int32, (DIM, DIM), 0) ==
           lax.broadcasted_iota(jnp.int32, (DIM, DIM), 1)).astype(jnp.float32)
    dn = (((0,), (0,)), ((), ()))
    pieces = []
    for i in range(VI):
        pieces.append(lax.dot_general(
            t_ref[0, :, i, :], eye, dn, preferred_element_type=jnp.float32))
    y_ref[...] = jnp.concatenate(pieces, axis=0)


@jax.jit
def _to_row_major_table(tbl_t4):
    return pl.pallas_call(
        _table_body,
        grid=(N_EMB, VOCAB // (VI * V2)),
        in_specs=[pl.BlockSpec((1, DIM, VI, V2),
                               lambda e, g: (e, 0, g, 0))],
        out_specs=pl.BlockSpec((VI * V2, DIM),
                               lambda e, g: (e * (VOCAB // (VI * V2)) + g, 0)),
        out_shape=jax.ShapeDtypeStruct((N_EMB * VOCAB, DIM), jnp.float32),
    )(tbl_t4)


def kernel(input_t, tables):
    s, b, e = input_t.shape
    idx_flat = input_t.astype(jnp.int32).reshape(-1)
    y = _to_row_major_table(
        tables.transpose(0, 2, 1).reshape(e, DIM, VOCAB // V2, V2))
    tbl_flat = y
    out = _multi_embedding(idx_flat, tbl_flat)
    x128 = out.reshape(NTOT * DIM // 128, 128)
    return _to_feature_major(x128).transpose(0, 2, 1)


# speedup vs baseline: 1.3396x; 1.3396x over previous
import functools

import jax
import jax.numpy as jnp
from jax import lax
from jax.experimental import pallas as pl
from jax.experimental.pallas import tpu as pltpu
from jax.experimental.pallas import tpu_sc as plsc

SEQ_LEN = 20
BATCH = 4096
N_EMB = 26
VOCAB = 100000
DIM = 32

NTOT = SEQ_LEN * BATCH * N_EMB
NW = 32
PER_W = NTOT // NW
PAT = 208
CHUNK = 832
SUB = 104
NSUB = CHUNK // SUB
NCHUNK = PER_W // CHUNK
assert PER_W % CHUNK == 0 and CHUNK % PAT == 0 and CHUNK % SUB == 0
assert NCHUNK >= 4 and NCHUNK % 2 == 0


def _body(idx_hbm, tbl_hbm, out_hbm, offs_v, idx_v0, idx_v1, rows_v0,
          rows_v1, isem0, isem1, gsem0, gsem1, osem0, osem1):
    wid = lax.axis_index("s") * 2 + lax.axis_index("c")
    idx_b = (idx_v0, idx_v1)
    rows_b = (rows_v0, rows_v1)
    isem = (isem0, isem1)
    gsem = (gsem0, gsem1)
    osem = (osem0, osem1)

    for i in range(PAT // 16):
        v = lax.iota(jnp.int32, 16) + (i * 16)
        offs_v[pl.ds(i * 16, 16)] = lax.rem(v, N_EMB) * VOCAB

    def base(c):
        return wid * PER_W + c * CHUNK

    def fire_gathers(b):
        for k in range(NSUB):
            pltpu.async_copy(
                tbl_hbm.at[idx_b[b].at[pl.ds(k * SUB, SUB)]],
                rows_b[b].at[pl.ds(k * SUB, SUB)],
                gsem[b])

    def drain_gathers(b):
        for k in range(NSUB):
            pltpu.make_async_copy(
                tbl_hbm.at[idx_b[b].at[pl.ds(k * SUB, SUB)]],
                rows_b[b].at[pl.ds(k * SUB, SUB)],
                gsem[b]).wait()

    def add_offsets(b):
        for j in range(CHUNK // 16):
            s = j * 16
            p = (j % (PAT // 16)) * 16
            idx_b[b][pl.ds(s, 16)] = (
                idx_b[b][pl.ds(s, 16)] + offs_v[pl.ds(p, 16)])

    def pipe_iter(c, b, has_prev, has_prev2, has_next):
        b1 = 1 - b
        if has_prev:
            drain_gathers(b1)
            pltpu.async_copy(rows_b[b1],
                             out_hbm.at[pl.ds(base(c - 1), CHUNK)], osem[b1])
        if has_next:
            pltpu.async_copy(idx_hbm.at[pl.ds(base(c + 1), CHUNK)],
                             idx_b[b1], isem[b1])
        pltpu.make_async_copy(idx_hbm.at[pl.ds(0, CHUNK)],
                              idx_b[b], isem[b]).wait()
        add_offsets(b)
        if has_prev2:
            pltpu.make_async_copy(rows_b[b], out_hbm.at[pl.ds(0, CHUNK)],
                                  osem[b]).wait()
        fire_gathers(b)

    pltpu.async_copy(idx_hbm.at[pl.ds(base(0), CHUNK)], idx_b[0], isem[0])
    pipe_iter(0, 0, False, False, True)
    pipe_iter(1, 1, True, False, True)

    def pair_body(t, carry):
        pipe_iter(2 * t, 0, True, True, True)
        pipe_iter(2 * t + 1, 1, True, True, True)
        return carry

    lax.fori_loop(1, NCHUNK // 2 - 1, pair_body, 0)

    pipe_iter(NCHUNK - 2, 0, True, True, True)
    pipe_iter(NCHUNK - 1, 1, True, True, False)

    drain_gathers(1)
    pltpu.async_copy(rows_b[1], out_hbm.at[pl.ds(base(NCHUNK - 1), CHUNK)],
                     osem[1])
    pltpu.make_async_copy(rows_b[0], out_hbm.at[pl.ds(0, CHUNK)],
                          osem[0]).wait()
    pltpu.make_async_copy(rows_b[1], out_hbm.at[pl.ds(0, CHUNK)],
                          osem[1]).wait()


@jax.jit
def _multi_embedding(idx_flat, tbl_flat):
    mesh = plsc.VectorSubcoreMesh(core_axis_name="c", subcore_axis_name="s")
    return pl.kernel(
        _body,
        out_type=jax.ShapeDtypeStruct((NTOT, DIM), jnp.float32),
        mesh=mesh,
        scratch_types=[
            pltpu.VMEM((PAT,), jnp.int32),
            pltpu.VMEM((CHUNK,), jnp.int32),
            pltpu.VMEM((CHUNK,), jnp.int32),
            pltpu.VMEM((CHUNK, DIM), jnp.float32),
            pltpu.VMEM((CHUNK, DIM), jnp.float32),
            pltpu.SemaphoreType.DMA,
            pltpu.SemaphoreType.DMA,
            pltpu.SemaphoreType.DMA,
            pltpu.SemaphoreType.DMA,
            pltpu.SemaphoreType.DMA,
            pltpu.SemaphoreType.DMA,
        ],
        compiler_params=pltpu.CompilerParams(use_tc_tiling_on_sc=False),
    )(idx_flat, tbl_flat)


FEAT = N_EMB * DIM


TCB = 8


def _transpose_body(x_ref, o_ref):
    m = lax.broadcasted_iota(jnp.int32, (64, 128), 0)
    c = lax.broadcasted_iota(jnp.int32, (64, 128), 1)
    p_ev = (c == 2 * m).astype(jnp.float32)
    p_od = (c == 2 * m + 1).astype(jnp.float32)
    dn = (((0,), (0,)), ((), ()))
    for k in range(TCB):
        x3 = x_ref[pl.ds(k * FEAT, FEAT), :].reshape(64, 13, 128)
        ev = jnp.concatenate(
            [x3[:, 0:6, :].reshape(64, 768), x3[:, 6, 0:64]], axis=1)
        od = jnp.concatenate(
            [x3[:, 6, 64:128], x3[:, 7:13, :].reshape(64, 768)], axis=1)
        o_ref[0, :, pl.ds(k * 128, 128)] = (
            lax.dot_general(ev, p_ev, dn, preferred_element_type=jnp.float32)
            + lax.dot_general(od, p_od, dn, preferred_element_type=jnp.float32))


@jax.jit
def _to_feature_major(x128):
    return pl.pallas_call(
        _transpose_body,
        grid=(SEQ_LEN, BATCH // (128 * TCB)),
        in_specs=[pl.BlockSpec((TCB * FEAT, 128),
                               lambda s, bb: (s * (BATCH // (128 * TCB)) + bb, 0))],
        out_specs=pl.BlockSpec((1, FEAT, TCB * 128), lambda s, bb: (s, 0, bb)),
        out_shape=jax.ShapeDtypeStruct((SEQ_LEN, FEAT, BATCH), jnp.float32),
    )(x128)


def kernel(input_t, tables):
    s, b, e = input_t.shape
    idx_flat = input_t.astype(jnp.int32).reshape(-1)
    tbl_flat = tables.reshape(e * VOCAB, DIM)
    out = _multi_embedding(idx_flat, tbl_flat)
    x128 = out.reshape(NTOT * DIM // 128, 128)
    return _to_feature_major(x128).transpose(0, 2, 1)


# speedup vs baseline: 1.3422x; 1.0020x over previous
import functools

import jax
import jax.numpy as jnp
from jax import lax
from jax.experimental import pallas as pl
from jax.experimental.pallas import tpu as pltpu
from jax.experimental.pallas import tpu_sc as plsc

SEQ_LEN = 20
BATCH = 4096
N_EMB = 26
VOCAB = 100000
DIM = 32

NTOT = SEQ_LEN * BATCH * N_EMB
NW = 32
PER_W = NTOT // NW
PAT = 208
CHUNK = 832
SUB = 104
NSUB = CHUNK // SUB
NCHUNK = PER_W // CHUNK
assert PER_W % CHUNK == 0 and CHUNK % PAT == 0 and CHUNK % SUB == 0
assert NCHUNK >= 4 and NCHUNK % 2 == 0


def _body(idx_hbm, tbl_hbm, out_hbm, offs_v, idx_v0, idx_v1, rows_v0,
          rows_v1, isem0, isem1, gsem0, gsem1, osem0, osem1):
    wid = lax.axis_index("s") * 2 + lax.axis_index("c")
    idx_b = (idx_v0, idx_v1)
    rows_b = (rows_v0, rows_v1)
    isem = (isem0, isem1)
    gsem = (gsem0, gsem1)
    osem = (osem0, osem1)

    for i in range(PAT // 16):
        v = lax.iota(jnp.int32, 16) + (i * 16)
        offs_v[pl.ds(i * 16, 16)] = lax.rem(v, N_EMB) * VOCAB

    def base(c):
        return wid * PER_W + c * CHUNK

    def fire_gathers(b):
        for k in range(NSUB):
            pltpu.async_copy(
                tbl_hbm.at[idx_b[b].at[pl.ds(k * SUB, SUB)]],
                rows_b[b].at[pl.ds(k * SUB, SUB)],
                gsem[b])

    def drain_gathers(b):
        for k in range(NSUB):
            pltpu.make_async_copy(
                tbl_hbm.at[idx_b[b].at[pl.ds(k * SUB, SUB)]],
                rows_b[b].at[pl.ds(k * SUB, SUB)],
                gsem[b]).wait()

    def add_offsets(b):
        for j in range(CHUNK // 16):
            s = j * 16
            p = (j % (PAT // 16)) * 16
            idx_b[b][pl.ds(s, 16)] = (
                idx_b[b][pl.ds(s, 16)] + offs_v[pl.ds(p, 16)])

    def pipe_iter(c, b, has_prev, has_prev2, has_next):
        b1 = 1 - b
        if has_prev:
            drain_gathers(b1)
            pltpu.async_copy(rows_b[b1],
                             out_hbm.at[pl.ds(base(c - 1), CHUNK)], osem[b1])
        if has_next:
            pltpu.async_copy(idx_hbm.at[pl.ds(base(c + 1), CHUNK)],
                             idx_b[b1], isem[b1])
        pltpu.make_async_copy(idx_hbm.at[pl.ds(0, CHUNK)],
                              idx_b[b], isem[b]).wait()
        add_offsets(b)
        if has_prev2:
            pltpu.make_async_copy(rows_b[b], out_hbm.at[pl.ds(0, CHUNK)],
                                  osem[b]).wait()
        fire_gathers(b)

    pltpu.async_copy(idx_hbm.at[pl.ds(base(0), CHUNK)], idx_b[0], isem[0])
    pipe_iter(0, 0, False, False, True)
    pipe_iter(1, 1, True, False, True)

    def pair_body(t, carry):
        pipe_iter(2 * t, 0, True, True, True)
        pipe_iter(2 * t + 1, 1, True, True, True)
        return carry

    lax.fori_loop(1, NCHUNK // 2 - 1, pair_body, 0)

    pipe_iter(NCHUNK - 2, 0, True, True, True)
    pipe_iter(NCHUNK - 1, 1, True, True, False)

    drain_gathers(1)
    pltpu.async_copy(rows_b[1], out_hbm.at[pl.ds(base(NCHUNK - 1), CHUNK)],
                     osem[1])
    pltpu.make_async_copy(rows_b[0], out_hbm.at[pl.ds(0, CHUNK)],
                          osem[0]).wait()
    pltpu.make_async_copy(rows_b[1], out_hbm.at[pl.ds(0, CHUNK)],
                          osem[1]).wait()


@jax.jit
def _multi_embedding(idx_flat, tbl_flat):
    mesh = plsc.VectorSubcoreMesh(core_axis_name="c", subcore_axis_name="s")
    return pl.kernel(
        _body,
        out_type=jax.ShapeDtypeStruct((NTOT, DIM), jnp.float32),
        mesh=mesh,
        scratch_types=[
            pltpu.VMEM((PAT,), jnp.int32),
            pltpu.VMEM((CHUNK,), jnp.int32),
            pltpu.VMEM((CHUNK,), jnp.int32),
            pltpu.VMEM((CHUNK, DIM), jnp.float32),
            pltpu.VMEM((CHUNK, DIM), jnp.float32),
            pltpu.SemaphoreType.DMA,
            pltpu.SemaphoreType.DMA,
            pltpu.SemaphoreType.DMA,
            pltpu.SemaphoreType.DMA,
            pltpu.SemaphoreType.DMA,
            pltpu.SemaphoreType.DMA,
        ],
        compiler_params=pltpu.CompilerParams(use_tc_tiling_on_sc=False),
    )(idx_flat, tbl_flat)


FEAT = N_EMB * DIM


TCB = 16


def _transpose_body(x_ref, o_ref):
    m = lax.broadcasted_iota(jnp.int32, (64, 128), 0)
    c = lax.broadcasted_iota(jnp.int32, (64, 128), 1)
    p_ev = (c == 2 * m).astype(jnp.float32)
    p_od = (c == 2 * m + 1).astype(jnp.float32)
    dn = (((0,), (0,)), ((), ()))
    for k in range(TCB):
        x3 = x_ref[pl.ds(k * FEAT, FEAT), :].reshape(64, 13, 128)
        ev = jnp.concatenate(
            [x3[:, 0:6, :].reshape(64, 768), x3[:, 6, 0:64]], axis=1)
        od = jnp.concatenate(
            [x3[:, 6, 64:128], x3[:, 7:13, :].reshape(64, 768)], axis=1)
        o_ref[0, :, pl.ds(k * 128, 128)] = (
            lax.dot_general(ev, p_ev, dn, preferred_element_type=jnp.float32)
            + lax.dot_general(od, p_od, dn, preferred_element_type=jnp.float32))


@jax.jit
def _to_feature_major(x128):
    return pl.pallas_call(
        _transpose_body,
        grid=(SEQ_LEN, BATCH // (128 * TCB)),
        in_specs=[pl.BlockSpec((TCB * FEAT, 128),
                               lambda s, bb: (s * (BATCH // (128 * TCB)) + bb, 0))],
        out_specs=pl.BlockSpec((1, FEAT, TCB * 128), lambda s, bb: (s, 0, bb)),
        out_shape=jax.ShapeDtypeStruct((SEQ_LEN, FEAT, BATCH), jnp.float32),
    )(x128)


def kernel(input_t, tables):
    s, b, e = input_t.shape
    idx_flat = input_t.astype(jnp.int32).reshape(-1)
    tbl_flat = tables.reshape(e * VOCAB, DIM)
    out = _multi_embedding(idx_flat, tbl_flat)
    x128 = out.reshape(NTOT * DIM // 128, 128)
    return _to_feature_major(x128).transpose(0, 2, 1)


# speedup vs baseline: 1.3433x; 1.0008x over previous
import jax
import jax.numpy as jnp
from jax import lax
from jax.experimental import pallas as pl
from jax.experimental.pallas import tpu as pltpu
from jax.experimental.pallas import tpu_sc as plsc

SEQ_LEN = 20
BATCH = 4096
N_EMB = 26
VOCAB = 100000
DIM = 32

NTOT = SEQ_LEN * BATCH * N_EMB
NW = 32
PER_W = NTOT // NW
PAT = 208
CHUNK = 832
SUB = 104
NSUB = CHUNK // SUB
NCHUNK = PER_W // CHUNK
assert PER_W % CHUNK == 0 and CHUNK % PAT == 0 and CHUNK % SUB == 0
assert NCHUNK >= 4 and NCHUNK % 2 == 0


def _body(idx_hbm, tbl_hbm, out_hbm, offs_v, idx_v0, idx_v1, rows_v0,
          rows_v1, isem0, isem1, gsem0, gsem1, osem0, osem1):
    wid = lax.axis_index("s") * 2 + lax.axis_index("c")
    idx_b = (idx_v0, idx_v1)
    rows_b = (rows_v0, rows_v1)
    isem = (isem0, isem1)
    gsem = (gsem0, gsem1)
    osem = (osem0, osem1)

    for i in range(PAT // 16):
        v = lax.iota(jnp.int32, 16) + (i * 16)
        offs_v[pl.ds(i * 16, 16)] = lax.rem(v, N_EMB) * VOCAB

    def base(c):
        return wid * PER_W + c * CHUNK

    def fire_gathers(b):
        for k in range(NSUB):
            pltpu.async_copy(
                tbl_hbm.at[idx_b[b].at[pl.ds(k * SUB, SUB)]],
                rows_b[b].at[pl.ds(k * SUB, SUB)],
                gsem[b])

    def drain_gathers(b):
        for k in range(NSUB):
            pltpu.make_async_copy(
                tbl_hbm.at[idx_b[b].at[pl.ds(k * SUB, SUB)]],
                rows_b[b].at[pl.ds(k * SUB, SUB)],
                gsem[b]).wait()

    def add_offsets(b):
        for j in range(CHUNK // 16):
            s = j * 16
            p = (j % (PAT // 16)) * 16
            idx_b[b][pl.ds(s, 16)] = (
                idx_b[b][pl.ds(s, 16)] + offs_v[pl.ds(p, 16)])

    def pipe_iter(c, b, has_prev, has_prev2, has_next):
        b1 = 1 - b
        if has_prev:
            drain_gathers(b1)
            pltpu.async_copy(rows_b[b1],
                             out_hbm.at[pl.ds(base(c - 1), CHUNK)], osem[b1])
        if has_next:
            pltpu.async_copy(idx_hbm.at[pl.ds(base(c + 1), CHUNK)],
                             idx_b[b1], isem[b1])
        pltpu.make_async_copy(idx_hbm.at[pl.ds(0, CHUNK)],
                              idx_b[b], isem[b]).wait()
        add_offsets(b)
        if has_prev2:
            pltpu.make_async_copy(rows_b[b], out_hbm.at[pl.ds(0, CHUNK)],
                                  osem[b]).wait()
        fire_gathers(b)

    pltpu.async_copy(idx_hbm.at[pl.ds(base(0), CHUNK)], idx_b[0], isem[0])
    pipe_iter(0, 0, False, False, True)
    pipe_iter(1, 1, True, False, True)

    def pair_body(t, carry):
        pipe_iter(2 * t, 0, True, True, True)
        pipe_iter(2 * t + 1, 1, True, True, True)
        return carry

    lax.fori_loop(1, NCHUNK // 2 - 1, pair_body, 0)

    pipe_iter(NCHUNK - 2, 0, True, True, True)
    pipe_iter(NCHUNK - 1, 1, True, True, False)

    drain_gathers(1)
    pltpu.async_copy(rows_b[1], out_hbm.at[pl.ds(base(NCHUNK - 1), CHUNK)],
                     osem[1])
    pltpu.make_async_copy(rows_b[0], out_hbm.at[pl.ds(0, CHUNK)],
                          osem[0]).wait()
    pltpu.make_async_copy(rows_b[1], out_hbm.at[pl.ds(0, CHUNK)],
                          osem[1]).wait()


@jax.jit
def _multi_embedding(idx_flat, tbl_flat):
    mesh = plsc.VectorSubcoreMesh(core_axis_name="c", subcore_axis_name="s")
    return pl.kernel(
        _body,
        out_type=jax.ShapeDtypeStruct((NTOT, DIM), jnp.float32),
        mesh=mesh,
        scratch_types=[
            pltpu.VMEM((PAT,), jnp.int32),
            pltpu.VMEM((CHUNK,), jnp.int32),
            pltpu.VMEM((CHUNK,), jnp.int32),
            pltpu.VMEM((CHUNK, DIM), jnp.float32),
            pltpu.VMEM((CHUNK, DIM), jnp.float32),
            pltpu.SemaphoreType.DMA,
            pltpu.SemaphoreType.DMA,
            pltpu.SemaphoreType.DMA,
            pltpu.SemaphoreType.DMA,
            pltpu.SemaphoreType.DMA,
            pltpu.SemaphoreType.DMA,
        ],
        compiler_params=pltpu.CompilerParams(use_tc_tiling_on_sc=False),
    )(idx_flat, tbl_flat)


FEAT = N_EMB * DIM


TCB = 16


def _transpose_body(x_ref, o_ref):
    m = lax.broadcasted_iota(jnp.int32, (64, 128), 0)
    c = lax.broadcasted_iota(jnp.int32, (64, 128), 1)
    p_ev = (c == 2 * m).astype(jnp.float32)
    p_od = (c == 2 * m + 1).astype(jnp.float32)
    dn = (((0,), (0,)), ((), ()))
    for k in range(TCB):
        x3 = x_ref[pl.ds(k * FEAT, FEAT), :].reshape(64, 13, 128)
        ev = jnp.concatenate(
            [x3[:, 0:6, :].reshape(64, 768), x3[:, 6, 0:64]], axis=1)
        od = jnp.concatenate(
            [x3[:, 6, 64:128], x3[:, 7:13, :].reshape(64, 768)], axis=1)
        o_ref[0, :, pl.ds(k * 128, 128)] = (
            lax.dot_general(ev, p_ev, dn, preferred_element_type=jnp.float32)
            + lax.dot_general(od, p_od, dn, preferred_element_type=jnp.float32))


@jax.jit
def _to_feature_major(x128):
    return pl.pallas_call(
        _transpose_body,
        grid=(SEQ_LEN, BATCH // (128 * TCB)),
        in_specs=[pl.BlockSpec((TCB * FEAT, 128),
                               lambda s, bb: (s * (BATCH // (128 * TCB)) + bb, 0))],
        out_specs=pl.BlockSpec((1, FEAT, TCB * 128), lambda s, bb: (s, 0, bb)),
        out_shape=jax.ShapeDtypeStruct((SEQ_LEN, FEAT, BATCH), jnp.float32),
    )(x128)


def kernel(input_t, tables):
    s, b, e = input_t.shape
    idx_flat = input_t.astype(jnp.int32).reshape(-1)
    tbl_flat = tables.reshape(e * VOCAB, DIM)
    out = _multi_embedding(idx_flat, tbl_flat)
    x128 = out.reshape(NTOT * DIM // 128, 128)
    return _to_feature_major(x128).transpose(0, 2, 1)


# speedup vs baseline: 1.3503x; 1.0052x over previous
import jax
import jax.numpy as jnp
from jax import lax
from jax.experimental import pallas as pl
from jax.experimental.pallas import tpu as pltpu
from jax.experimental.pallas import tpu_sc as plsc

SEQ_LEN = 20
BATCH = 4096
N_EMB = 26
VOCAB = 100000
DIM = 32

NTOT = SEQ_LEN * BATCH * N_EMB
NW = 32
PER_W = NTOT // NW
PAT = 208
CHUNK = 1664
SUB = 128
NSUB = CHUNK // SUB
NCHUNK = PER_W // CHUNK
assert PER_W % CHUNK == 0 and CHUNK % PAT == 0 and CHUNK % SUB == 0
assert NCHUNK >= 4 and NCHUNK % 2 == 0


def _body(idx_hbm, tbl_hbm, out_hbm, offs_v, idx_v0, idx_v1, rows_v0,
          rows_v1, isem0, isem1, gsem0, gsem1, osem0, osem1):
    wid = lax.axis_index("s") * 2 + lax.axis_index("c")
    idx_b = (idx_v0, idx_v1)
    rows_b = (rows_v0, rows_v1)
    isem = (isem0, isem1)
    gsem = (gsem0, gsem1)
    osem = (osem0, osem1)

    for i in range(PAT // 16):
        v = lax.iota(jnp.int32, 16) + (i * 16)
        offs_v[pl.ds(i * 16, 16)] = lax.rem(v, N_EMB) * VOCAB

    def base(c):
        return wid * PER_W + c * CHUNK

    def fire_gathers(b):
        for k in range(NSUB):
            pltpu.async_copy(
                tbl_hbm.at[idx_b[b].at[pl.ds(k * SUB, SUB)]],
                rows_b[b].at[pl.ds(k * SUB, SUB)],
                gsem[b])

    def drain_gathers(b):
        for k in range(NSUB):
            pltpu.make_async_copy(
                tbl_hbm.at[idx_b[b].at[pl.ds(k * SUB, SUB)]],
                rows_b[b].at[pl.ds(k * SUB, SUB)],
                gsem[b]).wait()

    def add_offsets(b):
        for j in range(CHUNK // 16):
            s = j * 16
            p = (j % (PAT // 16)) * 16
            idx_b[b][pl.ds(s, 16)] = (
                idx_b[b][pl.ds(s, 16)] + offs_v[pl.ds(p, 16)])

    def pipe_iter(c, b, has_prev, has_prev2, has_next):
        b1 = 1 - b
        if has_prev:
            drain_gathers(b1)
            pltpu.async_copy(rows_b[b1],
                             out_hbm.at[pl.ds(base(c - 1), CHUNK)], osem[b1])
        if has_next:
            pltpu.async_copy(idx_hbm.at[pl.ds(base(c + 1), CHUNK)],
                             idx_b[b1], isem[b1])
        pltpu.make_async_copy(idx_hbm.at[pl.ds(0, CHUNK)],
                              idx_b[b], isem[b]).wait()
        add_offsets(b)
        if has_prev2:
            pltpu.make_async_copy(rows_b[b], out_hbm.at[pl.ds(0, CHUNK)],
                                  osem[b]).wait()
        fire_gathers(b)

    pltpu.async_copy(idx_hbm.at[pl.ds(base(0), CHUNK)], idx_b[0], isem[0])
    pipe_iter(0, 0, False, False, True)
    pipe_iter(1, 1, True, False, True)

    def pair_body(t, carry):
        pipe_iter(2 * t, 0, True, True, True)
        pipe_iter(2 * t + 1, 1, True, True, True)
        return carry

    lax.fori_loop(1, NCHUNK // 2 - 1, pair_body, 0)

    pipe_iter(NCHUNK - 2, 0, True, True, True)
    pipe_iter(NCHUNK - 1, 1, True, True, False)

    drain_gathers(1)
    pltpu.async_copy(rows_b[1], out_hbm.at[pl.ds(base(NCHUNK - 1), CHUNK)],
                     osem[1])
    pltpu.make_async_copy(rows_b[0], out_hbm.at[pl.ds(0, CHUNK)],
                          osem[0]).wait()
    pltpu.make_async_copy(rows_b[1], out_hbm.at[pl.ds(0, CHUNK)],
                          osem[1]).wait()


@jax.jit
def _multi_embedding(idx_flat, tbl_flat):
    mesh = plsc.VectorSubcoreMesh(core_axis_name="c", subcore_axis_name="s")
    return pl.kernel(
        _body,
        out_type=jax.ShapeDtypeStruct((NTOT, DIM), jnp.float32),
        mesh=mesh,
        scratch_types=[
            pltpu.VMEM((PAT,), jnp.int32),
            pltpu.VMEM((CHUNK,), jnp.int32),
            pltpu.VMEM((CHUNK,), jnp.int32),
            pltpu.VMEM((CHUNK, DIM), jnp.float32),
            pltpu.VMEM((CHUNK, DIM), jnp.float32),
            pltpu.SemaphoreType.DMA,
            pltpu.SemaphoreType.DMA,
            pltpu.SemaphoreType.DMA,
            pltpu.SemaphoreType.DMA,
            pltpu.SemaphoreType.DMA,
            pltpu.SemaphoreType.DMA,
        ],
        compiler_params=pltpu.CompilerParams(use_tc_tiling_on_sc=False),
    )(idx_flat, tbl_flat)


FEAT = N_EMB * DIM


TCB = 16


def _transpose_body(x_ref, o_ref):
    m = lax.broadcasted_iota(jnp.int32, (64, 128), 0)
    c = lax.broadcasted_iota(jnp.int32, (64, 128), 1)
    p_ev = (c == 2 * m).astype(jnp.float32)
    p_od = (c == 2 * m + 1).astype(jnp.float32)
    dn = (((0,), (0,)), ((), ()))
    for k in range(TCB):
        x3 = x_ref[pl.ds(k * FEAT, FEAT), :].reshape(64, 13, 128)
        ev = jnp.concatenate(
            [x3[:, 0:6, :].reshape(64, 768), x3[:, 6, 0:64]], axis=1)
        od = jnp.concatenate(
            [x3[:, 6, 64:128], x3[:, 7:13, :].reshape(64, 768)], axis=1)
        o_ref[0, :, pl.ds(k * 128, 128)] = (
            lax.dot_general(ev, p_ev, dn, preferred_element_type=jnp.float32)
            + lax.dot_general(od, p_od, dn, preferred_element_type=jnp.float32))


@jax.jit
def _to_feature_major(x128):
    return pl.pallas_call(
        _transpose_body,
        grid=(SEQ_LEN, BATCH // (128 * TCB)),
        in_specs=[pl.BlockSpec((TCB * FEAT, 128),
                               lambda s, bb: (s * (BATCH // (128 * TCB)) + bb, 0))],
        out_specs=pl.BlockSpec((1, FEAT, TCB * 128), lambda s, bb: (s, 0, bb)),
        out_shape=jax.ShapeDtypeStruct((SEQ_LEN, FEAT, BATCH), jnp.float32),
    )(x128)


def kernel(input_t, tables):
    s, b, e = input_t.shape
    idx_flat = input_t.astype(jnp.int32).reshape(-1)
    tbl_flat = tables.reshape(e * VOCAB, DIM)
    out = _multi_embedding(idx_flat, tbl_flat)
    x128 = out.reshape(NTOT * DIM // 128, 128)
    return _to_feature_major(x128).transpose(0, 2, 1)
